# Initial kernel scaffold; baseline (speedup 1.0000x reference)
import jax
import jax.numpy as jnp
from jax.experimental import pallas as pl


def kernel(t, y, incidence, theta, bias):
    raise NotImplementedError("write your pallas kernel here")


# speedup vs baseline: 5.1624x; 5.1624x over previous
import jax
import jax.numpy as jnp
from jax import lax
from jax.experimental import pallas as pl
from jax.experimental.pallas import tpu as pltpu
from jax.experimental.pallas import tpu_sc as plsc

N_NODES = 10000
N_EDGES = 10000
NNZ = 320000
D = 128

CHUNK = 128
NCHUNKS = NNZ // CHUNK
NC = 2
NS = 16
NW = NC * NS
NPAD = 10240
ROWS_T = NPAD // NS
STRIP = 64

_BLK = 2048
_GRIDP = NPAD // _BLK
_OBLK = 2000
_GRIDO = N_NODES // _OBLK


def _project_body(y_ref, o_ref):
    y = y_ref[...]
    ss = jnp.sum(y * y, axis=1, keepdims=True)
    o_ref[...] = y / jnp.maximum(jnp.sqrt(ss), 1e-7)


def _combine_body(p_ref, o_ref):
    o_ref[...] = p_ref[0] + p_ref[1]


def _final_body(p_ref, th_ref, b_ref, o_ref):
    h = p_ref[0] + p_ref[1]
    acc = jnp.dot(h, th_ref[...], preferred_element_type=jnp.float32)
    o_ref[...] = jnp.tanh(acc + b_ref[...])


def _project(y):
    return pl.pallas_call(
        _project_body,
        grid=(N_NODES // _OBLK,),
        in_specs=[pl.BlockSpec((_OBLK, D), lambda i: (i, 0))],
        out_specs=pl.BlockSpec((_OBLK, D), lambda i: (i, 0)),
        out_shape=jax.ShapeDtypeStruct((N_NODES, D), jnp.float32),
    )(y)


def _combine(parts):
    return pl.pallas_call(
        _combine_body,
        grid=(_GRIDP,),
        in_specs=[pl.BlockSpec((NC, _BLK, D), lambda i: (0, i, 0))],
        out_specs=pl.BlockSpec((_BLK, D), lambda i: (i, 0)),
        out_shape=jax.ShapeDtypeStruct((NPAD, D), jnp.float32),
    )(parts)


def _final(parts, theta, bias):
    return pl.pallas_call(
        _final_body,
        grid=(_GRIDO,),
        in_specs=[
            pl.BlockSpec((NC, _OBLK, D), lambda i: (0, i, 0)),
            pl.BlockSpec((D, D), lambda i: (0, 0)),
            pl.BlockSpec((1, D), lambda i: (0, 0)),
        ],
        out_specs=pl.BlockSpec((_OBLK, D), lambda i: (i, 0)),
        out_shape=jax.ShapeDtypeStruct((N_NODES, D), jnp.float32),
    )(parts, theta, bias.reshape(1, D))


def _sc_mesh():
    return plsc.VectorSubcoreMesh(core_axis_name="c", subcore_axis_name="s")


def _make_agg_body(gather_row, scatter_row):
    def body(table_hbm, inc_hbm, zeros2d_hbm, zdeg_hbm, ones_hbm,
             parts_hbm,
             gidx_v, sidx_v, rows_v, ones_v, acc_v, degl_v,
             acc_sh, deg_sh):
        cid = lax.axis_index("c")
        sid = lax.axis_index("s")
        wid = sid * NC + cid
        r0 = sid * ROWS_T

        pltpu.sync_copy(ones_hbm, ones_v)

        pltpu.sync_copy(zeros2d_hbm, acc_sh.at[pl.ds(r0, ROWS_T)])
        pltpu.sync_copy(zdeg_hbm, deg_sh.at[pl.ds(r0, ROWS_T)])

        plsc.subcore_barrier()

        n_feat = (NCHUNKS - wid + NW - 1) // NW

        def feat_body(i, carry):
            base = (wid + i * NW) * CHUNK
            pltpu.sync_copy(inc_hbm.at[gather_row, pl.ds(base, CHUNK)], gidx_v)
            pltpu.sync_copy(inc_hbm.at[scatter_row, pl.ds(base, CHUNK)], sidx_v)
            pltpu.sync_copy(table_hbm.at[gidx_v], rows_v)
            pltpu.sync_copy(rows_v, acc_sh.at[sidx_v], add=True)
            return carry

        lax.fori_loop(0, n_feat, feat_body, 0)

        n_deg = (NCHUNKS - sid + NS - 1) // NS

        def deg_body(i, carry):
            base = (sid + i * NS) * CHUNK
            pltpu.sync_copy(inc_hbm.at[scatter_row, pl.ds(base, CHUNK)], sidx_v)
            pltpu.sync_copy(ones_v, deg_sh.at[sidx_v], add=True)
            return carry

        lax.fori_loop(0, n_deg, deg_body, 0)

        plsc.subcore_barrier()

        def emit_body(si, carry):
            sr0 = r0 + si * STRIP
            pltpu.sync_copy(acc_sh.at[pl.ds(sr0, STRIP)], acc_v)
            pltpu.sync_copy(deg_sh.at[pl.ds(sr0, STRIP)], degl_v)
            for g in range(STRIP // 16):
                dv = degl_v[pl.ds(16 * g, 16)]
                rec = 1.0 / jnp.maximum(dv, 1.0)
                for k in range(16):
                    r = 16 * g + k
                    s = rec[k]
                    for j in range(D // 16):
                        acc_v[r, pl.ds(16 * j, 16)] = acc_v[r, pl.ds(16 * j, 16)] * s
            pltpu.sync_copy(acc_v, parts_hbm.at[cid, pl.ds(sr0, STRIP)])
            return carry

        lax.fori_loop(0, ROWS_T // STRIP, emit_body, 0)

    return body


def _aggregate(table, incidence, gather_row, scatter_row,
               zeros2d, zdeg, ones):
    return pl.kernel(
        _make_agg_body(gather_row, scatter_row),
        out_type=jax.ShapeDtypeStruct((NC, NPAD, D), jnp.float32),
        mesh=_sc_mesh(),
        scratch_types=[
            pltpu.VMEM((CHUNK,), jnp.int32),
            pltpu.VMEM((CHUNK,), jnp.int32),
            pltpu.VMEM((CHUNK, D), jnp.float32),
            pltpu.VMEM((CHUNK,), jnp.float32),
            pltpu.VMEM((STRIP, D), jnp.float32),
            pltpu.VMEM((STRIP,), jnp.float32),
            pltpu.VMEM_SHARED((NPAD, D), jnp.float32),
            pltpu.VMEM_SHARED((NPAD,), jnp.float32),
        ],
    )(table, incidence, zeros2d, zdeg, ones)


@jax.jit
def kernel(t, y, incidence, theta, bias):
    del t
    zeros2d = jnp.zeros((ROWS_T, D), jnp.float32)
    zdeg = jnp.zeros((ROWS_T,), jnp.float32)
    ones = jnp.ones((CHUNK,), jnp.float32)

    y_proj = _project(y)
    edge_parts = _aggregate(y_proj, incidence, 0, 1, zeros2d, zdeg, ones)
    edge_feat = _combine(edge_parts)
    node_parts = _aggregate(edge_feat, incidence, 1, 0, zeros2d, zdeg, ones)
    return _final(node_parts, theta, bias)
